# baseline (speedup 1.0000x reference)
import functools

import jax
import jax.numpy as jnp
from jax.experimental import pallas as pl
from jax.experimental.pallas import tpu as pltpu

BATCH = 131072
BINS = 256
EPS = 0.01
BLOCK_ROWS = 512


def _tc_body(logits_ref, u_ref, out_ref):
    x = logits_ref[...]
    m = jnp.max(x, axis=-1, keepdims=True)
    e = jnp.exp(x - m)
    s = jnp.sum(e, axis=-1, keepdims=True)
    t = (1.0 - EPS) * e + (EPS / BINS) * s
    score = t / (-jnp.log(u_ref[...]))
    best = jnp.max(score, axis=-1, keepdims=True)
    iota = jax.lax.broadcasted_iota(jnp.int32, score.shape, 1)
    idx = jnp.min(jnp.where(score == best, iota, BINS), axis=-1, keepdims=True)
    probs = (1.0 - EPS) * (e / s) + EPS / BINS
    onehot = (iota == idx).astype(jnp.float32)
    out_ref[...] = onehot - jax.lax.stop_gradient(probs) + probs


def kernel(logits, u):
    grid = (BATCH // BLOCK_ROWS,)
    return pl.pallas_call(
        _tc_body,
        grid=grid,
        in_specs=[
            pl.BlockSpec((BLOCK_ROWS, BINS), lambda i: (i, 0)),
            pl.BlockSpec((BLOCK_ROWS, BINS), lambda i: (i, 0)),
        ],
        out_specs=pl.BlockSpec((BLOCK_ROWS, BINS), lambda i: (i, 0)),
        out_shape=jax.ShapeDtypeStruct((BATCH, BINS), jnp.float32),
    )(logits, u)


# speedup vs baseline: 1.5756x; 1.5756x over previous
import functools

import jax
import jax.numpy as jnp
from jax.experimental import pallas as pl
from jax.experimental.pallas import tpu as pltpu

BATCH = 131072
BINS = 256
EPS = 0.01
BLOCK_ROWS = 2048


def _tc_body(logits_ref, u_ref, out_ref):
    x = logits_ref[...]
    m = jnp.max(x, axis=-1, keepdims=True)
    e = jnp.exp(x - m)
    s = jnp.sum(e, axis=-1, keepdims=True)
    t = (1.0 - EPS) * e + (EPS / BINS) * s
    score = t / (-jnp.log(u_ref[...]))
    best = jnp.max(score, axis=-1, keepdims=True)
    iota = jax.lax.broadcasted_iota(jnp.int32, score.shape, 1)
    idx = jnp.min(jnp.where(score == best, iota, BINS), axis=-1, keepdims=True)
    probs = (1.0 - EPS) * (e / s) + EPS / BINS
    onehot = (iota == idx).astype(jnp.float32)
    out_ref[...] = onehot - jax.lax.stop_gradient(probs) + probs


def kernel(logits, u):
    grid = (BATCH // BLOCK_ROWS,)
    return pl.pallas_call(
        _tc_body,
        grid=grid,
        in_specs=[
            pl.BlockSpec((BLOCK_ROWS, BINS), lambda i: (i, 0)),
            pl.BlockSpec((BLOCK_ROWS, BINS), lambda i: (i, 0)),
        ],
        out_specs=pl.BlockSpec((BLOCK_ROWS, BINS), lambda i: (i, 0)),
        out_shape=jax.ShapeDtypeStruct((BATCH, BINS), jnp.float32),
    )(logits, u)


# speedup vs baseline: 1.7412x; 1.1051x over previous
import functools

import jax
import jax.numpy as jnp
from jax.experimental import pallas as pl
from jax.experimental.pallas import tpu as pltpu

BATCH = 131072
BINS = 256
EPS = 0.01
BLOCK_ROWS = 4096


def _tc_body(logits_ref, u_ref, out_ref):
    x = logits_ref[...]
    m = jnp.max(x, axis=-1, keepdims=True)
    e = jnp.exp(x - m)
    s = jnp.sum(e, axis=-1, keepdims=True)
    t = (1.0 - EPS) * e + (EPS / BINS) * s
    score = t / (-jnp.log(u_ref[...]))
    best = jnp.max(score, axis=-1, keepdims=True)
    iota = jax.lax.broadcasted_iota(jnp.int32, score.shape, 1)
    idx = jnp.min(jnp.where(score == best, iota, BINS), axis=-1, keepdims=True)
    probs = (1.0 - EPS) * (e / s) + EPS / BINS
    onehot = (iota == idx).astype(jnp.float32)
    out_ref[...] = onehot - jax.lax.stop_gradient(probs) + probs


def kernel(logits, u):
    grid = (BATCH // BLOCK_ROWS,)
    return pl.pallas_call(
        _tc_body,
        grid=grid,
        in_specs=[
            pl.BlockSpec((BLOCK_ROWS, BINS), lambda i: (i, 0)),
            pl.BlockSpec((BLOCK_ROWS, BINS), lambda i: (i, 0)),
        ],
        out_specs=pl.BlockSpec((BLOCK_ROWS, BINS), lambda i: (i, 0)),
        out_shape=jax.ShapeDtypeStruct((BATCH, BINS), jnp.float32),
    )(logits, u)
